# baseline (speedup 1.0000x reference)
import functools

import jax
import jax.numpy as jnp
from jax import lax
from jax.experimental import pallas as pl
from jax.experimental.pallas import tpu as pltpu
from jax.experimental.pallas import tpu_sc as plsc

_VOCAB = 100000
_EMB = 64
_B = 1024
_L = 50

_NC = 2
_NS = 16
_NW = _NC * _NS
_SAMP_PER_W = _B // _NW
_IDXW = 128
_NPASS = 4
_SPP = _SAMP_PER_W // _NPASS
_LG = 56
_TW = 128
_LANES = 16


def _pool_body(idx_hbm, table_hbm, out_hbm, idx_v, rows_v, out_v, sem):
    wid = lax.axis_index("s") * _NC + lax.axis_index("c")
    pltpu.sync_copy(
        idx_hbm.at[pl.ds(wid * _SAMP_PER_W, _SAMP_PER_W), pl.ds(0, _LG)], idx_v
    )
    scale = jnp.float32(1.0 / _L)

    def fire(p, buf):
        return [
            pltpu.async_copy(
                table_hbm.at[idx_v.at[p * _SPP + c]],
                rows_v.at[buf, pl.ds(c * _LG, _LG)],
                sem,
            )
            for c in range(_SPP)
        ]

    copies = fire(0, 0)
    for p in range(_NPASS):
        buf = p % 2
        for cp in copies:
            cp.wait()
        if p + 1 < _NPASS:
            copies = fire(p + 1, 1 - buf)

        def sample_body(ls, carry, p=p, buf=buf):
            base = ls * _LG
            acc = [jnp.zeros((_LANES,), jnp.float32) for _ in range(_EMB // _LANES)]
            for l in range(_L):
                r = base + l
                for k in range(_EMB // _LANES):
                    acc[k] = acc[k] + rows_v[buf, r, pl.ds(k * _LANES, _LANES)]
            s = p * _SPP + ls
            for k in range(_EMB // _LANES):
                out_v[s, pl.ds(k * _LANES, _LANES)] = acc[k] * scale
            return carry

        lax.fori_loop(0, _SPP, sample_body, jnp.int32(0))

    pltpu.sync_copy(out_v, out_hbm.at[pl.ds(wid * _SAMP_PER_W, _SAMP_PER_W)])


_pool = functools.partial(
    pl.kernel,
    out_type=jax.ShapeDtypeStruct((_B, _EMB), jnp.float32),
    mesh=plsc.VectorSubcoreMesh(core_axis_name="c", subcore_axis_name="s"),
    scratch_types=[
        pltpu.VMEM((_SAMP_PER_W, _LG), jnp.int32),
        pltpu.VMEM((2, _SPP * _LG, _TW), jnp.float32),
        pltpu.VMEM((_SAMP_PER_W, _EMB), jnp.float32),
        pltpu.SemaphoreType.DMA,
    ],
    compiler_params=pltpu.CompilerParams(use_tc_tiling_on_sc=False),
)(_pool_body)


_TBLK = 8192
_TGRID = (_VOCAB + _TBLK - 1) // _TBLK
_TTAIL = _VOCAB - (_TGRID - 1) * _TBLK
_TSPLIT = 4
_TRB = _TBLK // _TSPLIT
_TRT = _TTAIL // _TSPLIT


def _tr_body(e0, e1, e2, e3, out_ref, acc_ref, sem_ref):
    et_refs = (e0, e1, e2, e3)
    i = pl.program_id(0)
    slot = lax.rem(i, 2)

    def _out_copy(src_slot, blk, rows_per_copy):
        return [
            pltpu.make_async_copy(
                acc_ref.at[src_slot, pl.ds(r * rows_per_copy, rows_per_copy)],
                out_ref.at[pl.ds(blk * _TBLK + r * rows_per_copy, rows_per_copy)],
                sem_ref.at[src_slot, r],
            )
            for r in range(_TSPLIT)
        ]

    @pl.when(i >= 2)
    def _():
        for cp in _out_copy(slot, i - 2, _TRB):
            cp.wait()

    t = jnp.concatenate(
        [e[...].T for e in et_refs], axis=1
    )
    acc_ref[slot] = jnp.pad(t, ((0, 0), (0, _TW - _EMB)))

    @pl.when(i < _TGRID - 1)
    def _():
        for cp in _out_copy(slot, i, _TRB):
            cp.start()

    @pl.when(i == _TGRID - 1)
    def _():
        for cp in _out_copy(slot, i, _TRT):
            cp.start()
        for cp in _out_copy(1 - slot, i - 1, _TRB):
            cp.wait()
        for cp in _out_copy(slot, i, _TRT):
            cp.wait()


def _transpose_table(et):
    return pl.pallas_call(
        _tr_body,
        grid=(_TGRID,),
        in_specs=[
            pl.BlockSpec((_EMB // 4, _TBLK), (lambda i, r=r: (r, i)))
            for r in range(4)
        ],
        out_specs=pl.BlockSpec(memory_space=pl.ANY),
        out_shape=jax.ShapeDtypeStruct((_VOCAB, _TW), jnp.float32),
        scratch_shapes=[
            pltpu.VMEM((2, _TBLK, _TW), jnp.float32),
            pltpu.SemaphoreType.DMA((2, _TSPLIT)),
        ],
        compiler_params=pltpu.CompilerParams(
            dimension_semantics=("arbitrary",),
        ),
    )(et, et, et, et)


_NBLK = 4096
_GRID_N = (_VOCAB + _NBLK - 1) // _NBLK
_TAIL = _VOCAB - (_GRID_N - 1) * _NBLK
_RSPLIT = 8
_RB = _NBLK // _RSPLIT
_TSPLIT_TAIL = 4
_RT = _TAIL // _TSPLIT_TAIL


def _proj_body(x_ref, w_ref, b_ref, out_ref, acc_ref, sem_ref):
    i = pl.program_id(0)
    slot = lax.rem(i, 2)

    def _out_copy(src_slot, blk, rows_per_copy, n):
        copies = []
        for r in range(n):
            copies.append(
                pltpu.make_async_copy(
                    acc_ref.at[src_slot, pl.ds(r * rows_per_copy, rows_per_copy)],
                    out_ref.at[pl.ds(blk * _NBLK + r * rows_per_copy, rows_per_copy)],
                    sem_ref.at[src_slot, r],
                )
            )
        return copies

    @pl.when(i >= 2)
    def _():
        for cp in _out_copy(slot, i - 2, _RB, _RSPLIT):
            cp.wait()

    acc_ref[slot] = lax.dot_general(
        w_ref[...], x_ref[...],
        dimension_numbers=(((0,), (1,)), ((), ())),
        preferred_element_type=jnp.float32,
    ) + b_ref[...][:, None]

    @pl.when(i < _GRID_N - 1)
    def _():
        for cp in _out_copy(slot, i, _RB, _RSPLIT):
            cp.start()

    @pl.when(i == _GRID_N - 1)
    def _():
        for cp in _out_copy(slot, i, _RT, _TSPLIT_TAIL):
            cp.start()
        for cp in _out_copy(1 - slot, i - 1, _RB, _RSPLIT):
            cp.wait()
        for cp in _out_copy(slot, i, _RT, _TSPLIT_TAIL):
            cp.wait()


def _project(x, W, b):
    logits_t = pl.pallas_call(
        _proj_body,
        grid=(_GRID_N,),
        in_specs=[
            pl.BlockSpec((_B, _EMB), lambda i: (0, 0)),
            pl.BlockSpec((_EMB, _NBLK), lambda i: (0, i)),
            pl.BlockSpec((_NBLK,), lambda i: (i,)),
        ],
        out_specs=pl.BlockSpec(memory_space=pl.ANY),
        out_shape=jax.ShapeDtypeStruct((_VOCAB, _B), jnp.float32),
        scratch_shapes=[
            pltpu.VMEM((2, _NBLK, _B), jnp.float32),
            pltpu.SemaphoreType.DMA((2, _RSPLIT)),
        ],
        compiler_params=pltpu.CompilerParams(
            dimension_semantics=("arbitrary",),
        ),
    )(x, W, b)
    return logits_t.T


def kernel(inputs, emb_table, W, b):
    idx2d = jnp.concatenate([inputs, inputs, inputs[:, : _IDXW - 2 * _L]], axis=1)
    table2 = _transpose_table(emb_table.T)
    x = _pool(idx2d, table2)
    return _project(x, W, b)


# speedup vs baseline: 1.3846x; 1.3846x over previous
import functools

import jax
import jax.numpy as jnp
from jax import lax
from jax.experimental import pallas as pl
from jax.experimental.pallas import tpu as pltpu
from jax.experimental.pallas import tpu_sc as plsc

_VOCAB = 100000
_EMB = 64
_B = 1024
_L = 50

_NC = 2
_NS = 16
_NW = _NC * _NS
_SAMP_PER_W = _B // _NW
_IDXW = 128
_NPASS = 4
_SPP = _SAMP_PER_W // _NPASS
_LG = 56
_TW = 128
_LANES = 16


def _pool_body(idx_hbm, table_hbm, out_hbm, idx_v, rows_v, out_v, sem):
    wid = lax.axis_index("s") * _NC + lax.axis_index("c")
    pltpu.sync_copy(
        idx_hbm.at[pl.ds(wid * _SAMP_PER_W, _SAMP_PER_W), pl.ds(0, _LG)], idx_v
    )
    scale = jnp.float32(1.0 / _L)

    def fire(p, buf):
        return [
            pltpu.async_copy(
                table_hbm.at[idx_v.at[p * _SPP + c]],
                rows_v.at[buf, pl.ds(c * _LG, _LG)],
                sem,
            )
            for c in range(_SPP)
        ]

    copies = fire(0, 0)
    for p in range(_NPASS):
        buf = p % 2
        for cp in copies:
            cp.wait()
        if p + 1 < _NPASS:
            copies = fire(p + 1, 1 - buf)

        def sample_body(ls, carry, p=p, buf=buf):
            base = ls * _LG
            acc = [jnp.zeros((_LANES,), jnp.float32) for _ in range(_EMB // _LANES)]
            for l in range(_L):
                r = base + l
                for k in range(_EMB // _LANES):
                    acc[k] = acc[k] + rows_v[buf, r, pl.ds(k * _LANES, _LANES)]
            s = p * _SPP + ls
            for k in range(_EMB // _LANES):
                out_v[s, pl.ds(k * _LANES, _LANES)] = acc[k] * scale
            return carry

        lax.fori_loop(0, _SPP, sample_body, jnp.int32(0))

    pltpu.sync_copy(out_v, out_hbm.at[pl.ds(wid * _SAMP_PER_W, _SAMP_PER_W)])


_pool = functools.partial(
    pl.kernel,
    out_type=jax.ShapeDtypeStruct((_B, _EMB), jnp.float32),
    mesh=plsc.VectorSubcoreMesh(core_axis_name="c", subcore_axis_name="s"),
    scratch_types=[
        pltpu.VMEM((_SAMP_PER_W, _LG), jnp.int32),
        pltpu.VMEM((2, _SPP * _LG, _TW), jnp.float32),
        pltpu.VMEM((_SAMP_PER_W, _EMB), jnp.float32),
        pltpu.SemaphoreType.DMA,
    ],
    compiler_params=pltpu.CompilerParams(use_tc_tiling_on_sc=False),
)(_pool_body)


_TBLK = 8192
_TGRID = (_VOCAB + _TBLK - 1) // _TBLK
_TTAIL = _VOCAB - (_TGRID - 1) * _TBLK
_TSPLIT = 4
_TRB = _TBLK // _TSPLIT
_TRT = _TTAIL // _TSPLIT


def _tr_body(et_ref, out_ref, acc_ref, sem_ref):
    i = pl.program_id(0)
    slot = lax.rem(i, 2)

    def _out_copy(src_slot, blk, rows_per_copy):
        return [
            pltpu.make_async_copy(
                acc_ref.at[src_slot, pl.ds(r * rows_per_copy, rows_per_copy)],
                out_ref.at[pl.ds(blk * _TBLK + r * rows_per_copy, rows_per_copy)],
                sem_ref.at[src_slot, r],
            )
            for r in range(_TSPLIT)
        ]

    @pl.when(i >= 2)
    def _():
        for cp in _out_copy(slot, i - 2, _TRB):
            cp.wait()

    t = et_ref[...].T
    acc_ref[slot] = jnp.pad(t, ((0, 0), (0, _TW - _EMB)))

    @pl.when(i < _TGRID - 1)
    def _():
        for cp in _out_copy(slot, i, _TRB):
            cp.start()

    @pl.when(i == _TGRID - 1)
    def _():
        for cp in _out_copy(slot, i, _TRT):
            cp.start()
        for cp in _out_copy(1 - slot, i - 1, _TRB):
            cp.wait()
        for cp in _out_copy(slot, i, _TRT):
            cp.wait()


def _transpose_table(et):
    return pl.pallas_call(
        _tr_body,
        grid=(_TGRID,),
        in_specs=[pl.BlockSpec((_EMB, _TBLK), lambda i: (0, i))],
        out_specs=pl.BlockSpec(memory_space=pl.ANY),
        out_shape=jax.ShapeDtypeStruct((_VOCAB, _TW), jnp.float32),
        scratch_shapes=[
            pltpu.VMEM((2, _TBLK, _TW), jnp.float32),
            pltpu.SemaphoreType.DMA((2, _TSPLIT)),
        ],
        compiler_params=pltpu.CompilerParams(
            dimension_semantics=("arbitrary",),
        ),
    )(et)


_NBLK = 4096
_GRID_N = (_VOCAB + _NBLK - 1) // _NBLK
_TAIL = _VOCAB - (_GRID_N - 1) * _NBLK
_RSPLIT = 8
_RB = _NBLK // _RSPLIT
_TSPLIT_TAIL = 4
_RT = _TAIL // _TSPLIT_TAIL


def _proj_body(x_ref, w_ref, b_ref, out_ref, acc_ref, sem_ref):
    i = pl.program_id(0)
    slot = lax.rem(i, 2)

    def _out_copy(src_slot, blk, rows_per_copy, n):
        copies = []
        for r in range(n):
            copies.append(
                pltpu.make_async_copy(
                    acc_ref.at[src_slot, pl.ds(r * rows_per_copy, rows_per_copy)],
                    out_ref.at[pl.ds(blk * _NBLK + r * rows_per_copy, rows_per_copy)],
                    sem_ref.at[src_slot, r],
                )
            )
        return copies

    @pl.when(i >= 2)
    def _():
        for cp in _out_copy(slot, i - 2, _RB, _RSPLIT):
            cp.wait()

    acc_ref[slot] = lax.dot_general(
        w_ref[...], x_ref[...],
        dimension_numbers=(((0,), (1,)), ((), ())),
        preferred_element_type=jnp.float32,
    ) + b_ref[...][:, None]

    @pl.when(i < _GRID_N - 1)
    def _():
        for cp in _out_copy(slot, i, _RB, _RSPLIT):
            cp.start()

    @pl.when(i == _GRID_N - 1)
    def _():
        for cp in _out_copy(slot, i, _RT, _TSPLIT_TAIL):
            cp.start()
        for cp in _out_copy(1 - slot, i - 1, _RB, _RSPLIT):
            cp.wait()
        for cp in _out_copy(slot, i, _RT, _TSPLIT_TAIL):
            cp.wait()


def _project(x, W, b):
    logits_t = pl.pallas_call(
        _proj_body,
        grid=(_GRID_N,),
        in_specs=[
            pl.BlockSpec((_B, _EMB), lambda i: (0, 0)),
            pl.BlockSpec((_EMB, _NBLK), lambda i: (0, i)),
            pl.BlockSpec((_NBLK,), lambda i: (i,)),
        ],
        out_specs=pl.BlockSpec(memory_space=pl.ANY),
        out_shape=jax.ShapeDtypeStruct((_VOCAB, _B), jnp.float32),
        scratch_shapes=[
            pltpu.VMEM((2, _NBLK, _B), jnp.float32),
            pltpu.SemaphoreType.DMA((2, _RSPLIT)),
        ],
        compiler_params=pltpu.CompilerParams(
            dimension_semantics=("arbitrary",),
        ),
    )(x, W, b)
    return logits_t.T


def kernel(inputs, emb_table, W, b):
    idx2d = jnp.concatenate([inputs, inputs, inputs[:, : _IDXW - 2 * _L]], axis=1)
    table2 = _transpose_table(emb_table.T)
    x = _pool(idx2d, table2)
    return _project(x, W, b)
